# Initial kernel scaffold; baseline (speedup 1.0000x reference)
import jax
import jax.numpy as jnp
from jax.experimental import pallas as pl


def kernel(x, W, b):
    raise NotImplementedError("write your pallas kernel here")


# speedup vs baseline: 4.1525x; 4.1525x over previous
import functools

import jax
import jax.numpy as jnp
from jax import lax
from jax.experimental import pallas as pl
from jax.experimental.pallas import tpu as pltpu
from jax.experimental.pallas import tpu_sc as plsc

_B, _N, _D, _K, _F = 16, 2048, 64, 20, 256
_PD = 3
_RB = 256


def _tc_body(xa_ref, xr_ref, w_ref, idx_ref, g_ref):
    b = pl.program_id(0)
    xa = xa_ref[0]
    xr = xr_ref[0]
    pa = xa[:, :_PD]
    pr = xr[:, :_PD]

    cross = lax.dot_general(pr, pa, (((1,), (1,)), ((), ())),
                            preferred_element_type=jnp.float32)
    sqr = jnp.sum(pr * pr, axis=1, keepdims=True)
    sqa = jnp.sum(pa * pa, axis=1)[None, :]
    d = (sqr + sqa) - 2.0 * cross

    iota = lax.broadcasted_iota(jnp.int32, (_RB, _N), 1)
    lane_k = lax.broadcasted_iota(jnp.int32, (_RB, _K), 1)
    coll = jnp.zeros((_RB, _K), jnp.int32)
    base = b * _N
    big = jnp.float32(jnp.inf)
    for t in range(_K):
        m = jnp.min(d, axis=1, keepdims=True)
        amin = jnp.min(jnp.where(d == m, iota, _N), axis=1)
        d = jnp.where(iota == amin[:, None], big, d)
        coll = jnp.where(lane_k == t, amin[:, None] + base, coll)
    idx_ref[0] = coll

    g_ref[0] = jnp.dot(xr, w_ref[...], preferred_element_type=jnp.float32)


def _tc_call(x, W):
    return pl.pallas_call(
        _tc_body,
        grid=(_B, _N // _RB),
        in_specs=[
            pl.BlockSpec((1, _N, _D), lambda b, r: (b, 0, 0)),
            pl.BlockSpec((1, _RB, _D), lambda b, r: (b, r, 0)),
            pl.BlockSpec((_D, _F), lambda b, r: (0, 0)),
        ],
        out_specs=[
            pl.BlockSpec((1, _RB, _K), lambda b, r: (b, r, 0)),
            pl.BlockSpec((1, _RB, _F), lambda b, r: (b, r, 0)),
        ],
        out_shape=[
            jax.ShapeDtypeStruct((_B, _N, _K), jnp.int32),
            jax.ShapeDtypeStruct((_B, _N, _F), jnp.float32),
        ],
    )(x, x, W)


_P = _B * _N
_NW = 32
_PW = _P // _NW
_CP = 4
_NCH = _PW // _CP
_NFV = _F // 16
_NDV = _D // 16


def _sc_call(idx_flat, g_flat, x_flat, b):
    mesh = plsc.VectorSubcoreMesh(core_axis_name="c", subcore_axis_name="s")

    @functools.partial(
        pl.kernel,
        mesh=mesh,
        out_type=jax.ShapeDtypeStruct((_P, _F + _D), jnp.float32),
        scratch_types=[
            pltpu.VMEM((_CP * _K,), jnp.int32),
            pltpu.VMEM((_CP * _K, _F), jnp.float32),
            pltpu.VMEM((_CP, _F), jnp.float32),
            pltpu.VMEM((_CP, _D), jnp.float32),
            pltpu.VMEM((_CP, _F + _D), jnp.float32),
            pltpu.VMEM((_F,), jnp.float32),
            pltpu.SemaphoreType.DMA,
        ],
    )
    def sck(idx_hbm, g_hbm, x_hbm, b_hbm, out_hbm,
            idxv, rowsv, gsv, xv, outv, bv, sem):
        wid = lax.axis_index("s") * 2 + lax.axis_index("c")
        base0 = wid * _PW
        pltpu.sync_copy(b_hbm, bv)

        def body(ci, carry):
            pbase = base0 + ci * _CP
            pltpu.sync_copy(idx_hbm.at[pl.ds(pbase * _K, _CP * _K)], idxv)
            pltpu.async_copy(g_hbm.at[idxv], rowsv, sem).wait()
            pltpu.sync_copy(g_hbm.at[pl.ds(pbase, _CP)], gsv)
            pltpu.sync_copy(x_hbm.at[pl.ds(pbase, _CP)], xv)
            for j in range(_CP):
                for f in range(_NFV):
                    sl = pl.ds(f * 16, 16)
                    m = rowsv[j * _K, sl]
                    for r in range(1, _K):
                        m = jnp.maximum(m, rowsv[j * _K + r, sl])
                    gs = gsv[j, sl]
                    bb = bv[sl]
                    o = jnp.maximum(jnp.maximum(gs + bb, 0.0),
                                    jnp.maximum((m - gs) + bb, 0.0))
                    outv[j, sl] = o
                for k2 in range(_NDV):
                    outv[j, pl.ds(_F + k2 * 16, 16)] = xv[j, pl.ds(k2 * 16, 16)]
            pltpu.sync_copy(outv, out_hbm.at[pl.ds(pbase, _CP)])
            return carry

        lax.fori_loop(0, _NCH, body, 0)

    return sck(idx_flat, g_flat, x_flat, b)


def kernel(x, W, b):
    idx, g = _tc_call(x, W)
    idx_flat = idx.reshape(-1)
    g_flat = g.reshape(_P, _F)
    x_flat = x.reshape(_P, _D)
    out_flat = _sc_call(idx_flat, g_flat, x_flat, b)
    return out_flat.reshape(_B, _N, _F + _D)


# speedup vs baseline: 5.4817x; 1.3201x over previous
import functools

import jax
import jax.numpy as jnp
from jax import lax
from jax.experimental import pallas as pl
from jax.experimental.pallas import tpu as pltpu
from jax.experimental.pallas import tpu_sc as plsc

_B, _N, _D, _K, _F = 16, 2048, 64, 20, 256
_PD = 3
_RB = 256


def _tc_body(xa_ref, xr_ref, w_ref, idx_ref, g_ref):
    b = pl.program_id(0)
    xa = xa_ref[0]
    xr = xr_ref[0]
    pa = xa[:, :_PD]
    pr = xr[:, :_PD]

    cross = lax.dot_general(pr, pa, (((1,), (1,)), ((), ())),
                            preferred_element_type=jnp.float32)
    sqr = jnp.sum(pr * pr, axis=1, keepdims=True)
    sqa = jnp.sum(pa * pa, axis=1)[None, :]
    d = (sqr + sqa) - 2.0 * cross

    iota = lax.broadcasted_iota(jnp.int32, (_RB, _N), 1)
    lane_k = lax.broadcasted_iota(jnp.int32, (_RB, _K), 1)
    coll = jnp.zeros((_RB, _K), jnp.int32)
    base = b * _N
    big = jnp.float32(jnp.inf)
    for t in range(_K):
        m = jnp.min(d, axis=1, keepdims=True)
        amin = jnp.min(jnp.where(d == m, iota, _N), axis=1)
        d = jnp.where(iota == amin[:, None], big, d)
        coll = jnp.where(lane_k == t, amin[:, None] + base, coll)
    idx_ref[0] = coll

    g_ref[0] = jnp.dot(xr, w_ref[...], preferred_element_type=jnp.float32)


def _tc_call(x, W):
    return pl.pallas_call(
        _tc_body,
        grid=(_B, _N // _RB),
        in_specs=[
            pl.BlockSpec((1, _N, _D), lambda b, r: (b, 0, 0)),
            pl.BlockSpec((1, _RB, _D), lambda b, r: (b, r, 0)),
            pl.BlockSpec((_D, _F), lambda b, r: (0, 0)),
        ],
        out_specs=[
            pl.BlockSpec((1, _RB, _K), lambda b, r: (b, r, 0)),
            pl.BlockSpec((1, _RB, _F), lambda b, r: (b, r, 0)),
        ],
        out_shape=[
            jax.ShapeDtypeStruct((_B, _N, _K), jnp.int32),
            jax.ShapeDtypeStruct((_B, _N, _F), jnp.float32),
        ],
    )(x, x, W)


_P = _B * _N
_NW = 32
_PW = _P // _NW
_CP = 4
_NCH = _PW // _CP
_NFV = _F // 16
_NDV = _D // 16


def _sc_call(idx_flat, g_flat, x_flat, b):
    mesh = plsc.VectorSubcoreMesh(core_axis_name="c", subcore_axis_name="s")

    @functools.partial(
        pl.kernel,
        mesh=mesh,
        out_type=jax.ShapeDtypeStruct((_P, _F + _D), jnp.float32),
        scratch_types=[
            pltpu.VMEM((_PW * _K,), jnp.int32),
            pltpu.VMEM((_CP * _K, _F), jnp.float32),
            pltpu.VMEM((_CP * _K, _F), jnp.float32),
            pltpu.VMEM((_CP, _F), jnp.float32),
            pltpu.VMEM((_CP, _F), jnp.float32),
            pltpu.VMEM((_CP, _D), jnp.float32),
            pltpu.VMEM((_CP, _D), jnp.float32),
            pltpu.VMEM((_CP, _F + _D), jnp.float32),
            pltpu.VMEM((_CP, _F + _D), jnp.float32),
            pltpu.VMEM((_F,), jnp.float32),
            pltpu.SemaphoreType.DMA,
            pltpu.SemaphoreType.DMA,
            pltpu.SemaphoreType.DMA,
            pltpu.SemaphoreType.DMA,
        ],
    )
    def sck(idx_hbm, g_hbm, x_hbm, b_hbm, out_hbm,
            idxa, rows0, rows1, gs0, gs1, xv0, xv1, out0, out1, bv,
            semA, semB, semOutA, semOutB):
        wid = lax.axis_index("s") * 2 + lax.axis_index("c")
        base0 = wid * _PW
        pltpu.sync_copy(b_hbm, bv)
        pltpu.sync_copy(idx_hbm.at[pl.ds(base0 * _K, _PW * _K)], idxa)

        def issue_in(c, rowsv, gsv, xv, sem):
            pbase = base0 + c * _CP
            pltpu.async_copy(
                g_hbm.at[idxa.at[pl.ds(c * (_CP * _K), _CP * _K)]], rowsv, sem)
            pltpu.async_copy(g_hbm.at[pl.ds(pbase, _CP)], gsv, sem)
            pltpu.async_copy(x_hbm.at[pl.ds(pbase, _CP)], xv, sem)

        def wait_in(rowsv, gsv, xv, sem):
            pltpu.make_async_copy(g_hbm.at[pl.ds(0, _CP * _K)], rowsv, sem).wait()
            pltpu.make_async_copy(g_hbm.at[pl.ds(0, _CP)], gsv, sem).wait()
            pltpu.make_async_copy(x_hbm.at[pl.ds(0, _CP)], xv, sem).wait()

        def compute(rowsv, gsv, xv, outv):
            for j in range(_CP):
                for f in range(_NFV):
                    sl = pl.ds(f * 16, 16)
                    m = rowsv[j * _K, sl]
                    for r in range(1, _K):
                        m = jnp.maximum(m, rowsv[j * _K + r, sl])
                    gs = gsv[j, sl]
                    bb = bv[sl]
                    o = jnp.maximum(jnp.maximum(gs + bb, 0.0),
                                    jnp.maximum((m - gs) + bb, 0.0))
                    outv[j, sl] = o
                for k2 in range(_NDV):
                    outv[j, pl.ds(_F + k2 * 16, 16)] = xv[j, pl.ds(k2 * 16, 16)]

        def wait_out(outv, sem):
            pltpu.make_async_copy(outv, out_hbm.at[pl.ds(0, _CP)], sem).wait()

        issue_in(0, rows0, gs0, xv0, semA)

        def body(cj, carry):
            ca = 2 * cj
            cb = 2 * cj + 1
            issue_in(cb, rows1, gs1, xv1, semB)
            wait_in(rows0, gs0, xv0, semA)

            @pl.when(cj >= 1)
            def _():
                wait_out(out0, semOutA)
            compute(rows0, gs0, xv0, out0)
            pltpu.async_copy(out0, out_hbm.at[pl.ds(base0 + ca * _CP, _CP)],
                             semOutA)

            @pl.when(cb + 1 < _NCH)
            def _():
                issue_in(cb + 1, rows0, gs0, xv0, semA)
            wait_in(rows1, gs1, xv1, semB)

            @pl.when(cj >= 1)
            def _():
                wait_out(out1, semOutB)
            compute(rows1, gs1, xv1, out1)
            pltpu.async_copy(out1, out_hbm.at[pl.ds(base0 + cb * _CP, _CP)],
                             semOutB)
            return carry

        lax.fori_loop(0, _NCH // 2, body, 0)
        wait_out(out0, semOutA)
        wait_out(out1, semOutB)

    return sck(idx_flat, g_flat, x_flat, b)


def kernel(x, W, b):
    idx, g = _tc_call(x, W)
    idx_flat = idx.reshape(-1)
    g_flat = g.reshape(_P, _F)
    x_flat = x.reshape(_P, _D)
    out_flat = _sc_call(idx_flat, g_flat, x_flat, b)
    return out_flat.reshape(_B, _N, _F + _D)


# speedup vs baseline: 6.1815x; 1.1277x over previous
import functools

import jax
import jax.numpy as jnp
from jax import lax
from jax.experimental import pallas as pl
from jax.experimental.pallas import tpu as pltpu
from jax.experimental.pallas import tpu_sc as plsc

_B, _N, _D, _K, _F = 16, 2048, 64, 20, 256
_PD = 3
_RB = 256


def _tc_body(xa_ref, xr_ref, w_ref, idx_ref, g_ref):
    b = pl.program_id(0)
    xa = xa_ref[0]
    xr = xr_ref[0]
    pa = xa[:, :_PD]
    pr = xr[:, :_PD]

    cross = lax.dot_general(pr, pa, (((1,), (1,)), ((), ())),
                            preferred_element_type=jnp.float32)
    sqr = jnp.sum(pr * pr, axis=1, keepdims=True)
    sqa = jnp.sum(pa * pa, axis=1)[None, :]
    d = (sqr + sqa) - 2.0 * cross

    iota = lax.broadcasted_iota(jnp.int32, (_RB, _N), 1)
    lane_k = lax.broadcasted_iota(jnp.int32, (_RB, _K), 1)
    coll = jnp.zeros((_RB, _K), jnp.int32)
    base = b * _N
    big = jnp.float32(jnp.inf)
    for t in range(_K):
        m = jnp.min(d, axis=1, keepdims=True)
        eq = d == m
        amin = jnp.min(jnp.where(eq, iota, _N), axis=1)
        d = jnp.where(eq, big, d)
        coll = jnp.where(lane_k == t, amin[:, None] + base, coll)
    idx_ref[0] = coll

    g_ref[0] = jnp.dot(xr, w_ref[...], preferred_element_type=jnp.float32)


def _tc_call(x, W):
    return pl.pallas_call(
        _tc_body,
        grid=(_B, _N // _RB),
        in_specs=[
            pl.BlockSpec((1, _N, _D), lambda b, r: (b, 0, 0)),
            pl.BlockSpec((1, _RB, _D), lambda b, r: (b, r, 0)),
            pl.BlockSpec((_D, _F), lambda b, r: (0, 0)),
        ],
        out_specs=[
            pl.BlockSpec((1, _RB, _K), lambda b, r: (b, r, 0)),
            pl.BlockSpec((1, _RB, _F), lambda b, r: (b, r, 0)),
        ],
        out_shape=[
            jax.ShapeDtypeStruct((_B, _N, _K), jnp.int32),
            jax.ShapeDtypeStruct((_B, _N, _F), jnp.float32),
        ],
    )(x, x, W)


_P = _B * _N
_NW = 32
_PW = _P // _NW
_CP = 4
_NCH = _PW // _CP
_NFV = _F // 16
_NDV = _D // 16


def _sc_call(idx_flat, g_flat, x_flat, b):
    mesh = plsc.VectorSubcoreMesh(core_axis_name="c", subcore_axis_name="s")

    @functools.partial(
        pl.kernel,
        mesh=mesh,
        out_type=jax.ShapeDtypeStruct((_P, _F + _D), jnp.float32),
        scratch_types=[
            pltpu.VMEM((_PW * _K,), jnp.int32),
            pltpu.VMEM((_CP * _K, _F), jnp.float32),
            pltpu.VMEM((_CP * _K, _F), jnp.float32),
            pltpu.VMEM((_CP, _F), jnp.float32),
            pltpu.VMEM((_CP, _F), jnp.float32),
            pltpu.VMEM((_CP, _D), jnp.float32),
            pltpu.VMEM((_CP, _D), jnp.float32),
            pltpu.VMEM((_CP, _F + _D), jnp.float32),
            pltpu.VMEM((_CP, _F + _D), jnp.float32),
            pltpu.VMEM((_F,), jnp.float32),
            pltpu.SemaphoreType.DMA,
            pltpu.SemaphoreType.DMA,
            pltpu.SemaphoreType.DMA,
            pltpu.SemaphoreType.DMA,
        ],
    )
    def sck(idx_hbm, g_hbm, x_hbm, b_hbm, out_hbm,
            idxa, rows0, rows1, gs0, gs1, xv0, xv1, out0, out1, bv,
            semA, semB, semOutA, semOutB):
        wid = lax.axis_index("s") * 2 + lax.axis_index("c")
        base0 = wid * _PW
        pltpu.sync_copy(b_hbm, bv)
        pltpu.sync_copy(idx_hbm.at[pl.ds(base0 * _K, _PW * _K)], idxa)

        def issue_in(c, rowsv, gsv, xv, sem):
            pbase = base0 + c * _CP
            pltpu.async_copy(
                g_hbm.at[idxa.at[pl.ds(c * (_CP * _K), _CP * _K)]], rowsv, sem)
            pltpu.async_copy(g_hbm.at[pl.ds(pbase, _CP)], gsv, sem)
            pltpu.async_copy(x_hbm.at[pl.ds(pbase, _CP)], xv, sem)

        def wait_in(rowsv, gsv, xv, sem):
            pltpu.make_async_copy(g_hbm.at[pl.ds(0, _CP * _K)], rowsv, sem).wait()
            pltpu.make_async_copy(g_hbm.at[pl.ds(0, _CP)], gsv, sem).wait()
            pltpu.make_async_copy(x_hbm.at[pl.ds(0, _CP)], xv, sem).wait()

        def compute(rowsv, gsv, xv, outv):
            for j in range(_CP):
                for f in range(_NFV):
                    sl = pl.ds(f * 16, 16)
                    m = rowsv[j * _K, sl]
                    for r in range(1, _K):
                        m = jnp.maximum(m, rowsv[j * _K + r, sl])
                    gs = gsv[j, sl]
                    bb = bv[sl]
                    o = jnp.maximum(jnp.maximum(gs + bb, 0.0),
                                    jnp.maximum((m - gs) + bb, 0.0))
                    outv[j, sl] = o
                for k2 in range(_NDV):
                    outv[j, pl.ds(_F + k2 * 16, 16)] = xv[j, pl.ds(k2 * 16, 16)]

        def wait_out(outv, sem):
            pltpu.make_async_copy(outv, out_hbm.at[pl.ds(0, _CP)], sem).wait()

        issue_in(0, rows0, gs0, xv0, semA)

        def body(cj, carry):
            ca = 2 * cj
            cb = 2 * cj + 1
            issue_in(cb, rows1, gs1, xv1, semB)
            wait_in(rows0, gs0, xv0, semA)

            @pl.when(cj >= 1)
            def _():
                wait_out(out0, semOutA)
            compute(rows0, gs0, xv0, out0)
            pltpu.async_copy(out0, out_hbm.at[pl.ds(base0 + ca * _CP, _CP)],
                             semOutA)

            @pl.when(cb + 1 < _NCH)
            def _():
                issue_in(cb + 1, rows0, gs0, xv0, semA)
            wait_in(rows1, gs1, xv1, semB)

            @pl.when(cj >= 1)
            def _():
                wait_out(out1, semOutB)
            compute(rows1, gs1, xv1, out1)
            pltpu.async_copy(out1, out_hbm.at[pl.ds(base0 + cb * _CP, _CP)],
                             semOutB)
            return carry

        lax.fori_loop(0, _NCH // 2, body, 0)
        wait_out(out0, semOutA)
        wait_out(out1, semOutB)

    return sck(idx_flat, g_flat, x_flat, b)


def kernel(x, W, b):
    idx, g = _tc_call(x, W)
    idx_flat = idx.reshape(-1)
    g_flat = g.reshape(_P, _F)
    x_flat = x.reshape(_P, _D)
    out_flat = _sc_call(idx_flat, g_flat, x_flat, b)
    return out_flat.reshape(_B, _N, _F + _D)


# speedup vs baseline: 7.7058x; 1.2466x over previous
import functools

import jax
import jax.numpy as jnp
from jax import lax
from jax.experimental import pallas as pl
from jax.experimental.pallas import tpu as pltpu
from jax.experimental.pallas import tpu_sc as plsc

_B, _N, _D, _K, _F = 16, 2048, 64, 20, 256
_PD = 3
_RB = 256


def _tc_body(xa_ref, xr_ref, w_ref, idx_ref, g_ref):
    b = pl.program_id(0)
    xa = xa_ref[0]
    xr = xr_ref[0]
    pa = xa[:, :_PD]
    pr = xr[:, :_PD]

    cross = lax.dot_general(pr, pa, (((1,), (1,)), ((), ())),
                            preferred_element_type=jnp.float32)
    sqr = jnp.sum(pr * pr, axis=1, keepdims=True)
    sqa = jnp.sum(pa * pa, axis=1)[None, :]
    d = (sqr + sqa) - 2.0 * cross

    iota = lax.broadcasted_iota(jnp.int32, (_RB, _N), 1)
    lane_k = lax.broadcasted_iota(jnp.int32, (_RB, _K), 1)
    coll = jnp.zeros((_RB, _K), jnp.int32)
    base = b * _N
    big = jnp.float32(jnp.inf)
    for t in range(_K):
        m = jnp.min(d, axis=1, keepdims=True)
        eq = d == m
        amin = jnp.min(jnp.where(eq, iota, _N), axis=1)
        d = jnp.where(eq, big, d)
        coll = jnp.where(lane_k == t, amin[:, None] + base, coll)
    idx_ref[0] = coll

    g_ref[0] = jnp.dot(xr, w_ref[...], preferred_element_type=jnp.float32)


def _tc_call(x, W):
    nb = x.shape[0]
    return pl.pallas_call(
        _tc_body,
        grid=(nb, _N // _RB),
        in_specs=[
            pl.BlockSpec((1, _N, _D), lambda b, r: (b, 0, 0)),
            pl.BlockSpec((1, _RB, _D), lambda b, r: (b, r, 0)),
            pl.BlockSpec((_D, _F), lambda b, r: (0, 0)),
        ],
        out_specs=[
            pl.BlockSpec((1, _RB, _K), lambda b, r: (b, r, 0)),
            pl.BlockSpec((1, _RB, _F), lambda b, r: (b, r, 0)),
        ],
        out_shape=[
            jax.ShapeDtypeStruct((nb, _N, _K), jnp.int32),
            jax.ShapeDtypeStruct((nb, _N, _F), jnp.float32),
        ],
    )(x, x, W)


_P = _B * _N
_NW = 32
_PW = _P // _NW
_CP = 4
_NCH = _PW // _CP
_NFV = _F // 16
_NDV = _D // 16


def _sc_call(idx_flat, g_flat, x_flat, b):
    npts = g_flat.shape[0]
    pw = npts // _NW
    nch = pw // _CP
    mesh = plsc.VectorSubcoreMesh(core_axis_name="c", subcore_axis_name="s")

    @functools.partial(
        pl.kernel,
        mesh=mesh,
        out_type=jax.ShapeDtypeStruct((npts, _F + _D), jnp.float32),
        scratch_types=[
            pltpu.VMEM((pw * _K,), jnp.int32),
            pltpu.VMEM((_CP * _K, _F), jnp.float32),
            pltpu.VMEM((_CP * _K, _F), jnp.float32),
            pltpu.VMEM((_CP, _F), jnp.float32),
            pltpu.VMEM((_CP, _F), jnp.float32),
            pltpu.VMEM((_CP, _D), jnp.float32),
            pltpu.VMEM((_CP, _D), jnp.float32),
            pltpu.VMEM((_CP, _F + _D), jnp.float32),
            pltpu.VMEM((_CP, _F + _D), jnp.float32),
            pltpu.VMEM((_F,), jnp.float32),
            pltpu.SemaphoreType.DMA,
            pltpu.SemaphoreType.DMA,
            pltpu.SemaphoreType.DMA,
            pltpu.SemaphoreType.DMA,
        ],
    )
    def sck(idx_hbm, g_hbm, x_hbm, b_hbm, out_hbm,
            idxa, rows0, rows1, gs0, gs1, xv0, xv1, out0, out1, bv,
            semA, semB, semOutA, semOutB):
        wid = lax.axis_index("s") * 2 + lax.axis_index("c")
        base0 = wid * pw
        pltpu.sync_copy(b_hbm, bv)
        pltpu.sync_copy(idx_hbm.at[pl.ds(base0 * _K, pw * _K)], idxa)

        def issue_in(c, rowsv, gsv, xv, sem):
            pbase = base0 + c * _CP
            pltpu.async_copy(
                g_hbm.at[idxa.at[pl.ds(c * (_CP * _K), _CP * _K)]], rowsv, sem)
            pltpu.async_copy(g_hbm.at[pl.ds(pbase, _CP)], gsv, sem)
            pltpu.async_copy(x_hbm.at[pl.ds(pbase, _CP)], xv, sem)

        def wait_in(rowsv, gsv, xv, sem):
            pltpu.make_async_copy(g_hbm.at[pl.ds(0, _CP * _K)], rowsv, sem).wait()
            pltpu.make_async_copy(g_hbm.at[pl.ds(0, _CP)], gsv, sem).wait()
            pltpu.make_async_copy(x_hbm.at[pl.ds(0, _CP)], xv, sem).wait()

        def compute(rowsv, gsv, xv, outv):
            for j in range(_CP):
                for f in range(_NFV):
                    sl = pl.ds(f * 16, 16)
                    m = rowsv[j * _K, sl]
                    for r in range(1, _K):
                        m = jnp.maximum(m, rowsv[j * _K + r, sl])
                    gs = gsv[j, sl]
                    bb = bv[sl]
                    o = jnp.maximum(jnp.maximum(gs + bb, 0.0),
                                    jnp.maximum((m - gs) + bb, 0.0))
                    outv[j, sl] = o
                for k2 in range(_NDV):
                    outv[j, pl.ds(_F + k2 * 16, 16)] = xv[j, pl.ds(k2 * 16, 16)]

        def wait_out(outv, sem):
            pltpu.make_async_copy(outv, out_hbm.at[pl.ds(0, _CP)], sem).wait()

        issue_in(0, rows0, gs0, xv0, semA)

        def body(cj, carry):
            ca = 2 * cj
            cb = 2 * cj + 1
            issue_in(cb, rows1, gs1, xv1, semB)
            wait_in(rows0, gs0, xv0, semA)

            @pl.when(cj >= 1)
            def _():
                wait_out(out0, semOutA)
            compute(rows0, gs0, xv0, out0)
            pltpu.async_copy(out0, out_hbm.at[pl.ds(base0 + ca * _CP, _CP)],
                             semOutA)

            @pl.when(cb + 1 < nch)
            def _():
                issue_in(cb + 1, rows0, gs0, xv0, semA)
            wait_in(rows1, gs1, xv1, semB)

            @pl.when(cj >= 1)
            def _():
                wait_out(out1, semOutB)
            compute(rows1, gs1, xv1, out1)
            pltpu.async_copy(out1, out_hbm.at[pl.ds(base0 + cb * _CP, _CP)],
                             semOutB)
            return carry

        lax.fori_loop(0, nch // 2, body, 0)
        wait_out(out0, semOutA)
        wait_out(out1, semOutB)

    return sck(idx_flat, g_flat, x_flat, b)


def _half(x, W, b):
    nb = x.shape[0]
    npts = nb * _N
    idx, g = _tc_call(x, W)
    out_flat = _sc_call(idx.reshape(-1), g.reshape(npts, _F),
                        x.reshape(npts, _D), b)
    return out_flat.reshape(nb, _N, _F + _D)


def kernel(x, W, b):
    h = _B // 2
    return jnp.concatenate([_half(x[:h], W, b), _half(x[h:], W, b)], axis=0)


# speedup vs baseline: 8.7658x; 1.1375x over previous
import functools

import jax
import jax.numpy as jnp
from jax import lax
from jax.experimental import pallas as pl
from jax.experimental.pallas import tpu as pltpu
from jax.experimental.pallas import tpu_sc as plsc

_B, _N, _D, _K, _F = 16, 2048, 64, 20, 256
_PD = 3
_RB = 256


def _tc_body(xa_ref, xr_ref, w_ref, idx_ref, g_ref):
    b = pl.program_id(0)
    xa = xa_ref[0]
    xr = xr_ref[0]
    pa = xa[:, :_PD]
    pr = xr[:, :_PD]

    cross = lax.dot_general(pr, pa, (((1,), (1,)), ((), ())),
                            preferred_element_type=jnp.float32)
    sqr = jnp.sum(pr * pr, axis=1, keepdims=True)
    sqa = jnp.sum(pa * pa, axis=1)[None, :]
    d = (sqr + sqa) - 2.0 * cross

    iota = lax.broadcasted_iota(jnp.int32, (_RB, _N), 1)
    lane_k = lax.broadcasted_iota(jnp.int32, (_RB, _K), 1)
    coll = jnp.zeros((_RB, _K), jnp.int32)
    base = b * _N
    big = jnp.float32(jnp.inf)
    for t in range(_K):
        m = jnp.min(d, axis=1, keepdims=True)
        eq = d == m
        amin = jnp.min(jnp.where(eq, iota, _N), axis=1)
        d = jnp.where(eq, big, d)
        coll = jnp.where(lane_k == t, amin[:, None] + base, coll)
    idx_ref[0] = coll

    g_ref[0] = jnp.dot(xr, w_ref[...], preferred_element_type=jnp.float32)


def _tc_call(x, W):
    nb = x.shape[0]
    return pl.pallas_call(
        _tc_body,
        grid=(nb, _N // _RB),
        in_specs=[
            pl.BlockSpec((1, _N, _D), lambda b, r: (b, 0, 0)),
            pl.BlockSpec((1, _RB, _D), lambda b, r: (b, r, 0)),
            pl.BlockSpec((_D, _F), lambda b, r: (0, 0)),
        ],
        out_specs=[
            pl.BlockSpec((1, _RB, _K), lambda b, r: (b, r, 0)),
            pl.BlockSpec((1, _RB, _F), lambda b, r: (b, r, 0)),
        ],
        out_shape=[
            jax.ShapeDtypeStruct((nb, _N, _K), jnp.int32),
            jax.ShapeDtypeStruct((nb, _N, _F), jnp.float32),
        ],
    )(x, x, W)


_P = _B * _N
_NW = 32
_PW = _P // _NW
_CP = 4
_NCH = _PW // _CP
_NFV = _F // 16
_NDV = _D // 16


def _sc_call(idx_flat, g_flat, x_flat, b):
    npts = g_flat.shape[0]
    pw = npts // _NW
    nch = pw // _CP
    mesh = plsc.VectorSubcoreMesh(core_axis_name="c", subcore_axis_name="s")

    @functools.partial(
        pl.kernel,
        mesh=mesh,
        out_type=jax.ShapeDtypeStruct((npts, _F + _D), jnp.float32),
        scratch_types=[
            pltpu.VMEM((pw * _K,), jnp.int32),
            pltpu.VMEM((_CP * _K, _F), jnp.float32),
            pltpu.VMEM((_CP * _K, _F), jnp.float32),
            pltpu.VMEM((_CP, _F), jnp.float32),
            pltpu.VMEM((_CP, _F), jnp.float32),
            pltpu.VMEM((_CP, _D), jnp.float32),
            pltpu.VMEM((_CP, _D), jnp.float32),
            pltpu.VMEM((_CP, _F + _D), jnp.float32),
            pltpu.VMEM((_CP, _F + _D), jnp.float32),
            pltpu.VMEM((_F,), jnp.float32),
            pltpu.SemaphoreType.DMA,
            pltpu.SemaphoreType.DMA,
            pltpu.SemaphoreType.DMA,
            pltpu.SemaphoreType.DMA,
        ],
    )
    def sck(idx_hbm, g_hbm, x_hbm, b_hbm, out_hbm,
            idxa, rows0, rows1, gs0, gs1, xv0, xv1, out0, out1, bv,
            semA, semB, semOutA, semOutB):
        wid = lax.axis_index("s") * 2 + lax.axis_index("c")
        base0 = wid * pw
        pltpu.sync_copy(b_hbm, bv)
        pltpu.sync_copy(idx_hbm.at[pl.ds(base0 * _K, pw * _K)], idxa)

        def issue_in(c, rowsv, gsv, xv, sem):
            pbase = base0 + c * _CP
            pltpu.async_copy(
                g_hbm.at[idxa.at[pl.ds(c * (_CP * _K), _CP * _K)]], rowsv, sem)
            pltpu.async_copy(g_hbm.at[pl.ds(pbase, _CP)], gsv, sem)
            pltpu.async_copy(x_hbm.at[pl.ds(pbase, _CP)], xv, sem)

        def wait_in(rowsv, gsv, xv, sem):
            pltpu.make_async_copy(g_hbm.at[pl.ds(0, _CP * _K)], rowsv, sem).wait()
            pltpu.make_async_copy(g_hbm.at[pl.ds(0, _CP)], gsv, sem).wait()
            pltpu.make_async_copy(x_hbm.at[pl.ds(0, _CP)], xv, sem).wait()

        def compute(rowsv, gsv, xv, outv):
            for j in range(_CP):
                for f in range(_NFV):
                    sl = pl.ds(f * 16, 16)
                    m = rowsv[j * _K, sl]
                    for r in range(1, _K):
                        m = jnp.maximum(m, rowsv[j * _K + r, sl])
                    gs = gsv[j, sl]
                    bb = bv[sl]
                    o = jnp.maximum(jnp.maximum(gs + bb, 0.0),
                                    jnp.maximum((m - gs) + bb, 0.0))
                    outv[j, sl] = o
                for k2 in range(_NDV):
                    outv[j, pl.ds(_F + k2 * 16, 16)] = xv[j, pl.ds(k2 * 16, 16)]

        def wait_out(outv, sem):
            pltpu.make_async_copy(outv, out_hbm.at[pl.ds(0, _CP)], sem).wait()

        issue_in(0, rows0, gs0, xv0, semA)

        def body(cj, carry):
            ca = 2 * cj
            cb = 2 * cj + 1
            issue_in(cb, rows1, gs1, xv1, semB)
            wait_in(rows0, gs0, xv0, semA)

            @pl.when(cj >= 1)
            def _():
                wait_out(out0, semOutA)
            compute(rows0, gs0, xv0, out0)
            pltpu.async_copy(out0, out_hbm.at[pl.ds(base0 + ca * _CP, _CP)],
                             semOutA)

            @pl.when(cb + 1 < nch)
            def _():
                issue_in(cb + 1, rows0, gs0, xv0, semA)
            wait_in(rows1, gs1, xv1, semB)

            @pl.when(cj >= 1)
            def _():
                wait_out(out1, semOutB)
            compute(rows1, gs1, xv1, out1)
            pltpu.async_copy(out1, out_hbm.at[pl.ds(base0 + cb * _CP, _CP)],
                             semOutB)
            return carry

        lax.fori_loop(0, nch // 2, body, 0)
        wait_out(out0, semOutA)
        wait_out(out1, semOutB)

    return sck(idx_flat, g_flat, x_flat, b)


def _half(x, W, b):
    nb = x.shape[0]
    npts = nb * _N
    idx, g = _tc_call(x, W)
    out_flat = _sc_call(idx.reshape(-1), g.reshape(npts, _F),
                        x.reshape(npts, _D), b)
    return out_flat.reshape(nb, _N, _F + _D)


def kernel(x, W, b):
    h = _B // 4
    return jnp.concatenate(
        [_half(x[i * h:(i + 1) * h], W, b) for i in range(4)], axis=0)


# speedup vs baseline: 9.6407x; 1.0998x over previous
import functools

import jax
import jax.numpy as jnp
from jax import lax
from jax.experimental import pallas as pl
from jax.experimental.pallas import tpu as pltpu
from jax.experimental.pallas import tpu_sc as plsc

_B, _N, _D, _K, _F = 16, 2048, 64, 20, 256
_PD = 3
_RB = 256


def _tc_body(xa_ref, xr_ref, w_ref, idx_ref, g_ref):
    b = pl.program_id(0)
    xa = xa_ref[0]
    xr = xr_ref[0]
    pa = xa[:, :_PD]
    pr = xr[:, :_PD]

    cross = lax.dot_general(pr, pa, (((1,), (1,)), ((), ())),
                            preferred_element_type=jnp.float32)
    sqr = jnp.sum(pr * pr, axis=1, keepdims=True)
    sqa = jnp.sum(pa * pa, axis=1)[None, :]
    d = (sqr + sqa) - 2.0 * cross

    iota = lax.broadcasted_iota(jnp.int32, (_RB, _N), 1)
    lane_k = lax.broadcasted_iota(jnp.int32, (_RB, _K), 1)
    coll = jnp.zeros((_RB, _K), jnp.int32)
    base = b * _N
    big = jnp.float32(jnp.inf)
    pk = lax.bitcast_convert_type(
        ((lax.bitcast_convert_type(d, jnp.int32) + jnp.int32(64 << 23))
         & jnp.int32(~2047)) | iota,
        jnp.float32)
    for t in range(_K):
        m = jnp.min(d, axis=1, keepdims=True)
        eq = d == m
        mk = jnp.min(jnp.where(eq, pk, big), axis=1)
        amin = lax.bitcast_convert_type(mk, jnp.int32) & jnp.int32(2047)
        d = jnp.where(eq, big, d)
        coll = jnp.where(lane_k == t, amin[:, None] + base, coll)
    idx_ref[0] = coll

    g_ref[0] = jnp.dot(xr, w_ref[...], preferred_element_type=jnp.float32)


def _tc_call(x, W):
    nb = x.shape[0]
    return pl.pallas_call(
        _tc_body,
        grid=(nb, _N // _RB),
        in_specs=[
            pl.BlockSpec((1, _N, _D), lambda b, r: (b, 0, 0)),
            pl.BlockSpec((1, _RB, _D), lambda b, r: (b, r, 0)),
            pl.BlockSpec((_D, _F), lambda b, r: (0, 0)),
        ],
        out_specs=[
            pl.BlockSpec((1, _RB, _K), lambda b, r: (b, r, 0)),
            pl.BlockSpec((1, _RB, _F), lambda b, r: (b, r, 0)),
        ],
        out_shape=[
            jax.ShapeDtypeStruct((nb, _N, _K), jnp.int32),
            jax.ShapeDtypeStruct((nb, _N, _F), jnp.float32),
        ],
    )(x, x, W)


_P = _B * _N
_NW = 32
_PW = _P // _NW
_CP = 4
_NCH = _PW // _CP
_NFV = _F // 16
_NDV = _D // 16


def _sc_call(idx_flat, g_flat, x_flat, b):
    npts = g_flat.shape[0]
    pw = npts // _NW
    nch = pw // _CP
    mesh = plsc.VectorSubcoreMesh(core_axis_name="c", subcore_axis_name="s")

    @functools.partial(
        pl.kernel,
        mesh=mesh,
        out_type=jax.ShapeDtypeStruct((npts, _F + _D), jnp.float32),
        scratch_types=[
            pltpu.VMEM((pw * _K,), jnp.int32),
            pltpu.VMEM((_CP * _K, _F), jnp.float32),
            pltpu.VMEM((_CP * _K, _F), jnp.float32),
            pltpu.VMEM((_CP, _F), jnp.float32),
            pltpu.VMEM((_CP, _F), jnp.float32),
            pltpu.VMEM((_CP, _D), jnp.float32),
            pltpu.VMEM((_CP, _D), jnp.float32),
            pltpu.VMEM((_CP, _F + _D), jnp.float32),
            pltpu.VMEM((_CP, _F + _D), jnp.float32),
            pltpu.VMEM((_F,), jnp.float32),
            pltpu.SemaphoreType.DMA,
            pltpu.SemaphoreType.DMA,
            pltpu.SemaphoreType.DMA,
            pltpu.SemaphoreType.DMA,
        ],
    )
    def sck(idx_hbm, g_hbm, x_hbm, b_hbm, out_hbm,
            idxa, rows0, rows1, gs0, gs1, xv0, xv1, out0, out1, bv,
            semA, semB, semOutA, semOutB):
        wid = lax.axis_index("s") * 2 + lax.axis_index("c")
        base0 = wid * pw
        pltpu.sync_copy(b_hbm, bv)
        pltpu.sync_copy(idx_hbm.at[pl.ds(base0 * _K, pw * _K)], idxa)

        def issue_in(c, rowsv, gsv, xv, sem):
            pbase = base0 + c * _CP
            pltpu.async_copy(
                g_hbm.at[idxa.at[pl.ds(c * (_CP * _K), _CP * _K)]], rowsv, sem)
            pltpu.async_copy(g_hbm.at[pl.ds(pbase, _CP)], gsv, sem)
            pltpu.async_copy(x_hbm.at[pl.ds(pbase, _CP)], xv, sem)

        def wait_in(rowsv, gsv, xv, sem):
            pltpu.make_async_copy(g_hbm.at[pl.ds(0, _CP * _K)], rowsv, sem).wait()
            pltpu.make_async_copy(g_hbm.at[pl.ds(0, _CP)], gsv, sem).wait()
            pltpu.make_async_copy(x_hbm.at[pl.ds(0, _CP)], xv, sem).wait()

        def compute(rowsv, gsv, xv, outv):
            for j in range(_CP):
                for f in range(_NFV):
                    sl = pl.ds(f * 16, 16)
                    m = rowsv[j * _K, sl]
                    for r in range(1, _K):
                        m = jnp.maximum(m, rowsv[j * _K + r, sl])
                    gs = gsv[j, sl]
                    bb = bv[sl]
                    o = jnp.maximum(jnp.maximum(gs + bb, 0.0),
                                    jnp.maximum((m - gs) + bb, 0.0))
                    outv[j, sl] = o
                for k2 in range(_NDV):
                    outv[j, pl.ds(_F + k2 * 16, 16)] = xv[j, pl.ds(k2 * 16, 16)]

        def wait_out(outv, sem):
            pltpu.make_async_copy(outv, out_hbm.at[pl.ds(0, _CP)], sem).wait()

        issue_in(0, rows0, gs0, xv0, semA)

        def body(cj, carry):
            ca = 2 * cj
            cb = 2 * cj + 1
            issue_in(cb, rows1, gs1, xv1, semB)
            wait_in(rows0, gs0, xv0, semA)

            @pl.when(cj >= 1)
            def _():
                wait_out(out0, semOutA)
            compute(rows0, gs0, xv0, out0)
            pltpu.async_copy(out0, out_hbm.at[pl.ds(base0 + ca * _CP, _CP)],
                             semOutA)

            @pl.when(cb + 1 < nch)
            def _():
                issue_in(cb + 1, rows0, gs0, xv0, semA)
            wait_in(rows1, gs1, xv1, semB)

            @pl.when(cj >= 1)
            def _():
                wait_out(out1, semOutB)
            compute(rows1, gs1, xv1, out1)
            pltpu.async_copy(out1, out_hbm.at[pl.ds(base0 + cb * _CP, _CP)],
                             semOutB)
            return carry

        lax.fori_loop(0, nch // 2, body, 0)
        wait_out(out0, semOutA)
        wait_out(out1, semOutB)

    return sck(idx_flat, g_flat, x_flat, b)


def _half(x, W, b):
    nb = x.shape[0]
    npts = nb * _N
    idx, g = _tc_call(x, W)
    out_flat = _sc_call(idx.reshape(-1), g.reshape(npts, _F),
                        x.reshape(npts, _D), b)
    return out_flat.reshape(nb, _N, _F + _D)


def kernel(x, W, b):
    h = _B // 4
    return jnp.concatenate(
        [_half(x[i * h:(i + 1) * h], W, b) for i in range(4)], axis=0)
